# baseline (speedup 1.0000x reference)
import functools

import jax
import jax.numpy as jnp
from jax import lax
from jax.experimental import pallas as pl
from jax.experimental.pallas import tpu as pltpu
from jax.experimental.pallas import tpu_sc as plsc

N, E, D = 10000, 320000, 128
NC, NS = 2, 16
NW = NC * NS
CHUNK = 128
NCHUNKS = 80
EW = NCHUNKS * CHUNK
E_PAD = EW * NW
N_PAD = 10240
ROWS_PER_SUB = N_PAD // NS
DUMMY_ROW = N + 100


def _sc_aggregate(x, eidx):
    mesh = plsc.VectorSubcoreMesh(core_axis_name="c", subcore_axis_name="s")

    @functools.partial(
        pl.kernel,
        out_type=jax.ShapeDtypeStruct((NC, N_PAD, D), jnp.float32),
        mesh=mesh,
        scratch_types=[
            [pltpu.VMEM((2, CHUNK), jnp.int32) for _ in range(4)],
            [pltpu.VMEM((CHUNK, D), jnp.float32) for _ in range(2)],
            pltpu.VMEM_SHARED((N_PAD, D), jnp.float32),
            [pltpu.SemaphoreType.DMA for _ in range(4)],
            [pltpu.SemaphoreType.DMA for _ in range(2)],
        ],
    )
    def body(x_hbm, eidx_hbm, out_hbm, idx, rows, agg_sh, isem, gsem):
        core = lax.axis_index("c")
        sid = lax.axis_index("s")

        def fetch_idx(j, jm):
            pltpu.async_copy(eidx_hbm.at[core, sid, j], idx[jm], isem[jm])

        def wait_idx(j, jm):
            pltpu.make_async_copy(eidx_hbm.at[core, sid, j], idx[jm], isem[jm]).wait()

        def start_gather(jm, rm):
            pltpu.async_copy(x_hbm.at[idx[jm].at[0]], rows[rm], gsem[rm])

        def wait_gather(jm, rm):
            pltpu.make_async_copy(x_hbm.at[idx[jm].at[0]], rows[rm], gsem[rm]).wait()

        def zero_row(r, _):
            for cc in range(D // 16):
                rows[0][r, pl.ds(cc * 16, 16)] = jnp.zeros((16,), jnp.float32)
            return 0

        lax.fori_loop(0, CHUNK, zero_row, 0)
        for t in range(ROWS_PER_SUB // CHUNK):
            pltpu.sync_copy(rows[0], agg_sh.at[pl.ds(sid * ROWS_PER_SUB + t * CHUNK, CHUNK)])

        plsc.subcore_barrier()

        for j in range(3):
            fetch_idx(j, j)
        wait_idx(0, 0)
        start_gather(0, 0)

        def quad_body(q, _):
            for u in range(4):
                j = 4 * q + u
                jm, rm = u, u % 2
                fetch_idx(j + 3, (u + 3) % 4)
                wait_idx(j + 1, (u + 1) % 4)
                start_gather((u + 1) % 4, (u + 1) % 2)
                wait_gather(jm, rm)
                pltpu.sync_copy(rows[rm], agg_sh.at[idx[jm].at[1]], add=True)
            return 0

        lax.fori_loop(0, (NCHUNKS - 4) // 4, quad_body, 0)

        for j in range(NCHUNKS - 4, NCHUNKS):
            jm, rm = j % 4, j % 2
            if j + 3 < NCHUNKS:
                fetch_idx(j + 3, (j + 3) % 4)
            if j + 1 < NCHUNKS:
                wait_idx(j + 1, (j + 1) % 4)
                start_gather((j + 1) % 4, (j + 1) % 2)
            wait_gather(jm, rm)
            pltpu.sync_copy(rows[rm], agg_sh.at[idx[jm].at[1]], add=True)

        plsc.subcore_barrier()

        pltpu.sync_copy(
            agg_sh.at[pl.ds(sid * ROWS_PER_SUB, ROWS_PER_SUB)],
            out_hbm.at[core, pl.ds(sid * ROWS_PER_SUB, ROWS_PER_SUB)],
        )

    return body(x, eidx)


def _tc_linear(x, agg0, agg1, w, b2):
    BLK = 2000

    def body(x_ref, a0_ref, a1_ref, w_ref, b_ref, out_ref):
        h = x_ref[...] + a0_ref[...] + a1_ref[...]
        acc = lax.dot_general(
            h, w_ref[...], (((1,), (1,)), ((), ())),
            preferred_element_type=jnp.float32,
        )
        out_ref[...] = acc + b_ref[...]

    return pl.pallas_call(
        body,
        grid=(N // BLK,),
        in_specs=[
            pl.BlockSpec((BLK, D), lambda i: (i, 0)),
            pl.BlockSpec((BLK, D), lambda i: (i, 0)),
            pl.BlockSpec((BLK, D), lambda i: (i, 0)),
            pl.BlockSpec((D, D), lambda i: (0, 0)),
            pl.BlockSpec((1, D), lambda i: (0, 0)),
        ],
        out_specs=pl.BlockSpec((BLK, D), lambda i: (i, 0)),
        out_shape=jax.ShapeDtypeStruct((N, D), jnp.float32),
    )(x, agg0, agg1, w, b2)


@jax.jit
def kernel(node_inputs, edge_index, W, b):
    src = edge_index[0].astype(jnp.int32)
    dst = edge_index[1].astype(jnp.int32)
    pad = E_PAD - E
    src_p = jnp.concatenate([src, jnp.zeros((pad,), jnp.int32)])
    dst_p = jnp.concatenate([dst, jnp.full((pad,), DUMMY_ROW, jnp.int32)])
    eidx = jnp.stack(
        [src_p.reshape(NC, NS, NCHUNKS, CHUNK),
         dst_p.reshape(NC, NS, NCHUNKS, CHUNK)], axis=3)

    agg = _sc_aggregate(node_inputs, eidx)
    return _tc_linear(node_inputs, agg[0], agg[1], W, b.reshape(1, D))


# speedup vs baseline: 1.0011x; 1.0011x over previous
import functools

import jax
import jax.numpy as jnp
from jax import lax
from jax.experimental import pallas as pl
from jax.experimental.pallas import tpu as pltpu
from jax.experimental.pallas import tpu_sc as plsc

N, E, D = 10000, 320000, 128
NC, NS = 2, 16
NW = NC * NS
CHUNK = 128
NPH = 2
PCH = 40
NCHUNKS = NPH * PCH
EW = NCHUNKS * CHUNK
E_PAD = EW * NW
N_PAD = 10240
ROWS_PER_SUB = N_PAD // NS
DUMMY_ROW = N + 100


def _sc_aggregate(x, src_slab, dst_slab):
    mesh = plsc.VectorSubcoreMesh(core_axis_name="c", subcore_axis_name="s")

    @functools.partial(
        pl.kernel,
        out_type=jax.ShapeDtypeStruct((NC, N_PAD, D), jnp.float32),
        mesh=mesh,
        scratch_types=[
            pltpu.VMEM((PCH, CHUNK), jnp.int32),
            pltpu.VMEM((PCH, CHUNK), jnp.int32),
            [pltpu.VMEM((CHUNK, D), jnp.float32) for _ in range(2)],
            pltpu.VMEM_SHARED((N_PAD, D), jnp.float32),
            [pltpu.SemaphoreType.DMA for _ in range(2)],
        ],
    )
    def body(x_hbm, src_hbm, dst_hbm, out_hbm, src_v, dst_v, rows, agg_sh, gsem):
        core = lax.axis_index("c")
        sid = lax.axis_index("s")

        def start_gather(j, u):
            pltpu.async_copy(x_hbm.at[src_v.at[j]], rows[u], gsem[u])

        def wait_gather(j, u):
            pltpu.make_async_copy(x_hbm.at[src_v.at[j]], rows[u], gsem[u]).wait()

        def scatter(j, u):
            pltpu.sync_copy(rows[u], agg_sh.at[dst_v.at[j]], add=True)

        def zero_row(r, _):
            for cc in range(D // 16):
                rows[0][r, pl.ds(cc * 16, 16)] = jnp.zeros((16,), jnp.float32)
            return 0

        lax.fori_loop(0, CHUNK, zero_row, 0)
        for t in range(ROWS_PER_SUB // CHUNK):
            pltpu.sync_copy(rows[0], agg_sh.at[pl.ds(sid * ROWS_PER_SUB + t * CHUNK, CHUNK)])

        plsc.subcore_barrier()

        for ph in range(NPH):
            pltpu.sync_copy(src_hbm.at[core, sid, ph], src_v)
            pltpu.sync_copy(dst_hbm.at[core, sid, ph], dst_v)
            for u in range(2):
                start_gather(u, u)

            def pair_body(p, _):
                for u in range(2):
                    j = 2 * p + u
                    wait_gather(j, u)
                    scatter(j, u)
                    start_gather(j + 2, u)
                return 0

            lax.fori_loop(0, PCH // 2 - 1, pair_body, 0)
            for u in range(2):
                j = PCH - 2 + u
                wait_gather(j, u)
                scatter(j, u)

        plsc.subcore_barrier()

        pltpu.sync_copy(
            agg_sh.at[pl.ds(sid * ROWS_PER_SUB, ROWS_PER_SUB)],
            out_hbm.at[core, pl.ds(sid * ROWS_PER_SUB, ROWS_PER_SUB)],
        )

    return body(x, src_slab, dst_slab)


def _tc_linear(x, agg0, agg1, w, b2):
    BLK = 2000

    def body(x_ref, a0_ref, a1_ref, w_ref, b_ref, out_ref):
        h = x_ref[...] + a0_ref[...] + a1_ref[...]
        acc = lax.dot_general(
            h, w_ref[...], (((1,), (1,)), ((), ())),
            preferred_element_type=jnp.float32,
        )
        out_ref[...] = acc + b_ref[...]

    return pl.pallas_call(
        body,
        grid=(N // BLK,),
        in_specs=[
            pl.BlockSpec((BLK, D), lambda i: (i, 0)),
            pl.BlockSpec((BLK, D), lambda i: (i, 0)),
            pl.BlockSpec((BLK, D), lambda i: (i, 0)),
            pl.BlockSpec((D, D), lambda i: (0, 0)),
            pl.BlockSpec((1, D), lambda i: (0, 0)),
        ],
        out_specs=pl.BlockSpec((BLK, D), lambda i: (i, 0)),
        out_shape=jax.ShapeDtypeStruct((N, D), jnp.float32),
    )(x, agg0, agg1, w, b2)


@jax.jit
def kernel(node_inputs, edge_index, W, b):
    src = edge_index[0].astype(jnp.int32)
    dst = edge_index[1].astype(jnp.int32)
    pad = E_PAD - E
    src_p = jnp.concatenate([src, jnp.zeros((pad,), jnp.int32)])
    dst_p = jnp.concatenate([dst, jnp.full((pad,), DUMMY_ROW, jnp.int32)])
    src_slab = src_p.reshape(NC, NS, NPH, PCH, CHUNK)
    dst_slab = dst_p.reshape(NC, NS, NPH, PCH, CHUNK)

    agg = _sc_aggregate(node_inputs, src_slab, dst_slab)
    return _tc_linear(node_inputs, agg[0], agg[1], W, b.reshape(1, D))
